# Initial kernel scaffold; baseline (speedup 1.0000x reference)
import jax
import jax.numpy as jnp
from jax.experimental import pallas as pl


def kernel(xyz, batch_indices, semantics, robot_pose, robot_heading):
    raise NotImplementedError("write your pallas kernel here")


# speedup vs baseline: 2.0856x; 2.0856x over previous
import functools

import jax
import jax.numpy as jnp
from jax import lax
from jax.experimental import pallas as pl
from jax.experimental.pallas import tpu as pltpu
from jax.experimental.pallas import tpu_sc as plsc

_B = 16
_NUM_CLASSES = 20
_NUM_ROWS = 240
_NUM_COLS = 240
_MAPB = _NUM_CLASSES * _NUM_ROWS * _NUM_COLS
_N = 1_000_000
_NP = 1_024_000
_PER_TILE = _NP // 16
_K = 2_000
_NCHUNK = _PER_TILE // _K
_SLICE = _MAPB // 16
_ZB = 18_000
_MAGIC = 12582912.0


def _body(px_in, py_in, pz_in, bi, se, hx, hy, hz, cs, sn, out,
          thx, thy, thz, tc, ts, xb, yb, zb, bib, seb, idxb, valb, buf,
          smap):
    cid = lax.axis_index("c")
    sid = lax.axis_index("s")

    pltpu.sync_copy(hx, thx)
    pltpu.sync_copy(hy, thy)
    pltpu.sync_copy(hz, thz)
    pltpu.sync_copy(cs, tc)
    pltpu.sync_copy(sn, ts)

    def pass_body(k, _):
        b = cid * 8 + k

        def zfill(i, _):
            buf[pl.ds(i * 16, 16)] = jnp.zeros((16,), jnp.float32)
            return 0
        lax.fori_loop(0, _ZB // 16, zfill, 0)
        for q in range(4):
            pltpu.sync_copy(buf, smap.at[pl.ds(sid * _SLICE + q * _ZB, _ZB)])
        plsc.subcore_barrier()

        def chunk_body(j, _):
            base = sid * _PER_TILE + j * _K
            pltpu.sync_copy(px_in.at[pl.ds(base, _K)], xb)
            pltpu.sync_copy(py_in.at[pl.ds(base, _K)], yb)
            pltpu.sync_copy(pz_in.at[pl.ds(base, _K)], zb)
            pltpu.sync_copy(bi.at[pl.ds(base, _K)], bib)
            pltpu.sync_copy(se.at[pl.ds(base, _K)], seb)

            def vbody(v, _):
                sl = pl.ds(v * 16, 16)
                xv = xb[sl]
                yv = yb[sl]
                zv = zb[sl]
                biv = bib[sl]
                sev = seb[sl]
                hxv = plsc.load_gather(thx, [biv])
                hyv = plsc.load_gather(thy, [biv])
                hzv = plsc.load_gather(thz, [biv])
                cv = plsc.load_gather(tc, [biv])
                sv = plsc.load_gather(ts, [biv])
                hm = (yv > hyv - 1.25) & (yv < hyv + 0.75)
                p0 = xv - hxv
                p1 = yv - hyv
                p2 = zv - hzv
                pxv = cv * p0 + sv * p2
                pzv = (-sv) * p0 + cv * p2
                rf = (pzv + 12.0) / 0.1
                cf = (pxv + 12.0) / 0.1
                ri = ((rf + _MAGIC) - _MAGIC).astype(jnp.int32)
                ci = ((cf + _MAGIC) - _MAGIC).astype(jnp.int32)
                ok = (hm & (ri >= 0) & (ri < _NUM_ROWS)
                      & (ci >= 0) & (ci < _NUM_COLS) & (biv == b))
                ric = jnp.clip(ri, 0, _NUM_ROWS - 1)
                cic = jnp.clip(ci, 0, _NUM_COLS - 1)
                idxb[sl] = sev * (_NUM_ROWS * _NUM_COLS) + ric * _NUM_COLS + cic
                valb[sl] = jnp.where(ok, p1, 0.0)
                return 0
            lax.fori_loop(0, _K // 16, vbody, 0)
            pltpu.sync_copy(valb, smap.at[idxb], add=True)
            return 0
        lax.fori_loop(0, _NCHUNK, chunk_body, 0)
        plsc.subcore_barrier()
        for q in range(4):
            pltpu.sync_copy(smap.at[pl.ds(sid * _SLICE + q * _ZB, _ZB)], buf)
            pltpu.sync_copy(
                buf, out.at[pl.ds(b * _MAPB + sid * _SLICE + q * _ZB, _ZB)])
        plsc.subcore_barrier()
        return 0
    lax.fori_loop(0, 8, pass_body, 0)


_sc_call = pl.kernel(
    _body,
    out_type=jax.ShapeDtypeStruct((_B * _MAPB,), jnp.float32),
    mesh=plsc.VectorSubcoreMesh(core_axis_name="c", subcore_axis_name="s",
                                num_cores=2, num_subcores=16),
    compiler_params=pltpu.CompilerParams(needs_layout_passes=False),
    scratch_types=[
        pltpu.VMEM((16,), jnp.float32),
        pltpu.VMEM((16,), jnp.float32),
        pltpu.VMEM((16,), jnp.float32),
        pltpu.VMEM((16,), jnp.float32),
        pltpu.VMEM((16,), jnp.float32),
        pltpu.VMEM((_K,), jnp.float32),
        pltpu.VMEM((_K,), jnp.float32),
        pltpu.VMEM((_K,), jnp.float32),
        pltpu.VMEM((_K,), jnp.int32),
        pltpu.VMEM((_K,), jnp.int32),
        pltpu.VMEM((_K,), jnp.int32),
        pltpu.VMEM((_K,), jnp.float32),
        pltpu.VMEM((_ZB,), jnp.float32),
        pltpu.VMEM_SHARED((_MAPB,), jnp.float32),
    ],
)


def kernel(xyz, batch_indices, semantics, robot_pose, robot_heading):
    pad = _NP - _N
    xt = jnp.pad(jnp.transpose(xyz), ((0, 0), (0, pad)))
    bi = jnp.pad(batch_indices.astype(jnp.int32), (0, pad),
                 constant_values=_B)
    se = jnp.pad(semantics.astype(jnp.int32), (0, pad))
    ang = -robot_heading
    out = _sc_call(xt[0], xt[1], xt[2], bi, se,
                   robot_pose[:, 0], robot_pose[:, 1], robot_pose[:, 2],
                   jnp.cos(ang), jnp.sin(ang))
    return out.reshape(_B, _NUM_CLASSES, _NUM_ROWS, _NUM_COLS)
